# baseline (speedup 1.0000x reference)
import functools

import jax
import jax.numpy as jnp
from jax import lax
from jax.experimental import pallas as pl
from jax.experimental.pallas import tpu as pltpu
from jax.experimental.pallas import tpu_sc as plsc

VOCAB = 30522
HID = 768
L = 40
IMG = 384
PATCH = 16
CH = 3
GRID = IMG // PATCH
NPATCH = GRID * GRID
NTOK = 1 + NPATCH
EPS = 1e-12

_NC = 2
_NS = 16
_NW = _NC * _NS


def _sc_gather_fn(n_ids, rows_per_w):
    mesh = plsc.VectorSubcoreMesh(core_axis_name="c", subcore_axis_name="s")

    @functools.partial(
        pl.kernel,
        mesh=mesh,
        out_type=jax.ShapeDtypeStruct((n_ids, HID), jnp.float32),
        scratch_types=[
            pltpu.VMEM((rows_per_w,), jnp.int32),
            pltpu.VMEM((rows_per_w, HID), jnp.float32),
            pltpu.SemaphoreType.DMA,
        ],
    )
    def gather_kernel(table_hbm, idx_hbm, out_hbm, idx_v, rows_v, sem):
        wid = lax.axis_index("s") * _NC + lax.axis_index("c")
        base = wid * rows_per_w
        pltpu.sync_copy(idx_hbm.at[pl.ds(base, rows_per_w)], idx_v)
        pltpu.async_copy(table_hbm.at[idx_v], rows_v, sem).wait()
        pltpu.sync_copy(rows_v, out_hbm.at[pl.ds(base, rows_per_w)])

    return gather_kernel


def _tc_body(we_ref, pos_ref, tok_ref, mod_ref, g_ref, b_ref,
             w_ref, pb_ref, cls_ref, ipos_ref, px_ref, out_ref):
    tb = we_ref[0] + pos_ref[...] + tok_ref[0:1, :]
    m = jnp.mean(tb, axis=-1, keepdims=True)
    v = jnp.mean((tb - m) ** 2, axis=-1, keepdims=True)
    tn = (tb - m) * lax.rsqrt(v + EPS) * g_ref[...] + b_ref[...]
    out_ref[0, 0:L, :] = tn + mod_ref[0:1, :]

    px = px_ref[0].astype(jnp.bfloat16)
    acc = None
    for c in range(CH):
        pc = (px[c].reshape(GRID, PATCH, GRID, PATCH)
              .transpose(0, 2, 1, 3)
              .reshape(NPATCH, PATCH * PATCH))
        d = jnp.dot(pc, w_ref[c * PATCH * PATCH:(c + 1) * PATCH * PATCH, :],
                    preferred_element_type=jnp.float32)
        acc = d if acc is None else acc + d
    img = acc + pb_ref[...] + ipos_ref[1:NTOK, :] + mod_ref[1:2, :]
    cls_row = cls_ref[...] + ipos_ref[0:1, :] + mod_ref[1:2, :]
    out_ref[0, L:L + NTOK, :] = jnp.concatenate([cls_row, img], axis=0)


def kernel(input_ids, attention_mask, token_type_ids, pixel_values, pixel_mask,
           word_emb, pos_emb, tok_type_emb, ln_g, ln_b,
           patch_W, patch_b, cls_token, img_pos_emb, mod_type_emb):
    B, Lx = input_ids.shape
    n_ids = B * Lx
    rows_per_w = n_ids // _NW

    ids_flat = input_ids.reshape(n_ids).astype(jnp.int32)
    we = _sc_gather_fn(n_ids, rows_per_w)(word_emb, ids_flat)
    we = we.reshape(B, Lx, HID)

    px6 = pixel_values.reshape(B, CH, GRID, PATCH, GRID, PATCH)

    full = lambda shape: pl.BlockSpec(shape, lambda b: (0,) * len(shape))
    out = pl.pallas_call(
        _tc_body,
        grid=(B,),
        in_specs=[
            pl.BlockSpec((1, Lx, HID), lambda b: (b, 0, 0)),
            full((Lx, HID)),
            full((1, HID)),
            full((2, HID)),
            full((1, HID)),
            full((1, HID)),
            full((HID, HID)),
            full((1, HID)),
            full((1, HID)),
            full((NTOK, HID)),
            pl.BlockSpec((1, CH, IMG, IMG), lambda b: (b, 0, 0, 0)),
        ],
        out_specs=pl.BlockSpec((1, Lx + NTOK, HID), lambda b: (b, 0, 0)),
        out_shape=jax.ShapeDtypeStruct((B, Lx + NTOK, HID), jnp.float32),
    )(we, pos_emb[:Lx], tok_type_emb[0:1], mod_type_emb,
      ln_g.reshape(1, HID), ln_b.reshape(1, HID),
      patch_W.astype(jnp.bfloat16), patch_b.reshape(1, HID),
      cls_token.reshape(1, HID), img_pos_emb, pixel_values)

    masks = jnp.concatenate(
        [attention_mask,
         jnp.ones((B, NTOK), dtype=attention_mask.dtype)], axis=1)
    return (out, masks)


# speedup vs baseline: 1.0198x; 1.0198x over previous
import functools

import jax
import jax.numpy as jnp
from jax import lax
from jax.experimental import pallas as pl
from jax.experimental.pallas import tpu as pltpu
from jax.experimental.pallas import tpu_sc as plsc

VOCAB = 30522
HID = 768
L = 40
IMG = 384
PATCH = 16
CH = 3
GRID = IMG // PATCH
NPATCH = GRID * GRID
NTOK = 1 + NPATCH
EPS = 1e-12
_BB = 2

_NC = 2
_NS = 16
_NW = _NC * _NS


def _sc_gather_fn(n_ids, rows_per_w):
    mesh = plsc.VectorSubcoreMesh(core_axis_name="c", subcore_axis_name="s")

    @functools.partial(
        pl.kernel,
        mesh=mesh,
        out_type=jax.ShapeDtypeStruct((n_ids, HID), jnp.float32),
        scratch_types=[
            pltpu.VMEM((rows_per_w,), jnp.int32),
            pltpu.VMEM((rows_per_w, HID), jnp.float32),
            pltpu.SemaphoreType.DMA,
        ],
    )
    def gather_kernel(table_hbm, idx_hbm, out_hbm, idx_v, rows_v, sem):
        wid = lax.axis_index("s") * _NC + lax.axis_index("c")
        base = wid * rows_per_w
        pltpu.sync_copy(idx_hbm.at[pl.ds(base, rows_per_w)], idx_v)
        pltpu.async_copy(table_hbm.at[idx_v], rows_v, sem).wait()
        pltpu.sync_copy(rows_v, out_hbm.at[pl.ds(base, rows_per_w)])

    return gather_kernel


def _tc_body(we_ref, pos_ref, tok_ref, mod_ref, g_ref, b_ref,
             w_ref, pb_ref, cls_ref, ipos_ref, px_ref, out_ref):
    for i in range(_BB):
        tb = we_ref[i] + pos_ref[...] + tok_ref[0:1, :]
        m = jnp.mean(tb, axis=-1, keepdims=True)
        v = jnp.mean((tb - m) ** 2, axis=-1, keepdims=True)
        tn = (tb - m) * lax.rsqrt(v + EPS) * g_ref[...] + b_ref[...]
        out_ref[i, 0:L, :] = tn + mod_ref[0:1, :]

        px = px_ref[i].astype(jnp.bfloat16)
        acc = None
        for c in range(CH):
            pc = (px[c].reshape(GRID, PATCH, GRID, PATCH)
                  .transpose(0, 2, 1, 3)
                  .reshape(NPATCH, PATCH * PATCH))
            d = jnp.dot(pc, w_ref[c * PATCH * PATCH:(c + 1) * PATCH * PATCH, :],
                        preferred_element_type=jnp.float32)
            acc = d if acc is None else acc + d
        img = acc + pb_ref[...] + ipos_ref[1:NTOK, :] + mod_ref[1:2, :]
        cls_row = cls_ref[...] + ipos_ref[0:1, :] + mod_ref[1:2, :]
        out_ref[i, L:L + NTOK, :] = jnp.concatenate([cls_row, img], axis=0)


def kernel(input_ids, attention_mask, token_type_ids, pixel_values, pixel_mask,
           word_emb, pos_emb, tok_type_emb, ln_g, ln_b,
           patch_W, patch_b, cls_token, img_pos_emb, mod_type_emb):
    B, Lx = input_ids.shape
    n_ids = B * Lx
    rows_per_w = n_ids // _NW

    ids_flat = input_ids.reshape(n_ids).astype(jnp.int32)
    we = _sc_gather_fn(n_ids, rows_per_w)(word_emb, ids_flat)
    we = we.reshape(B, Lx, HID)

    full = lambda shape: pl.BlockSpec(shape, lambda b: (0,) * len(shape))
    out = pl.pallas_call(
        _tc_body,
        grid=(B // _BB,),
        in_specs=[
            pl.BlockSpec((_BB, Lx, HID), lambda b: (b, 0, 0)),
            full((Lx, HID)),
            full((1, HID)),
            full((2, HID)),
            full((1, HID)),
            full((1, HID)),
            full((HID, HID)),
            full((1, HID)),
            full((1, HID)),
            full((NTOK, HID)),
            pl.BlockSpec((_BB, CH, IMG, IMG), lambda b: (b, 0, 0, 0)),
        ],
        out_specs=pl.BlockSpec((_BB, Lx + NTOK, HID), lambda b: (b, 0, 0)),
        out_shape=jax.ShapeDtypeStruct((B, Lx + NTOK, HID), jnp.float32),
    )(we, pos_emb[:Lx], tok_type_emb[0:1], mod_type_emb,
      ln_g.reshape(1, HID), ln_b.reshape(1, HID),
      patch_W.astype(jnp.bfloat16), patch_b.reshape(1, HID),
      cls_token.reshape(1, HID), img_pos_emb, pixel_values)

    masks = jnp.concatenate(
        [attention_mask,
         jnp.ones((B, NTOK), dtype=attention_mask.dtype)], axis=1)
    return (out, masks)
